# baseline (speedup 1.0000x reference)
import functools

import jax
import jax.numpy as jnp
from jax import lax
from jax.experimental import pallas as pl
from jax.experimental.pallas import tpu as pltpu
from jax.experimental.pallas import tpu_sc as plsc

N = 10000
M = 320000
D = 128
NC = 2
NS = 16
NW = NC * NS
MW = M // NW
CW = 80
CHUNKS = MW // CW
NP = 10240
RPT = NP // NS

_mesh = plsc.VectorSubcoreMesh(
    core_axis_name="c", subcore_axis_name="s", num_cores=NC, num_subcores=NS)

F32 = jnp.float32


def _prep_body(v_ref, w1a_ref, w1b_ref, b1_ref, va_ref, vb_ref):
    v = v_ref[...]
    va_ref[...] = jnp.dot(v, w1a_ref[...], preferred_element_type=F32)
    vb_ref[...] = jnp.dot(v, w1b_ref[...], preferred_element_type=F32) + b1_ref[...]


def _prep(v, w1a, w1b, b1):
    bn = 1000
    grid = N // bn
    return pl.pallas_call(
        _prep_body,
        grid=(grid,),
        in_specs=[
            pl.BlockSpec((bn, D), lambda j: (j, 0)),
            pl.BlockSpec((D, D), lambda j: (0, 0)),
            pl.BlockSpec((D, D), lambda j: (0, 0)),
            pl.BlockSpec((1, D), lambda j: (0, 0)),
        ],
        out_specs=[
            pl.BlockSpec((bn, D), lambda j: (j, 0)),
            pl.BlockSpec((bn, D), lambda j: (j, 0)),
        ],
        out_shape=[jax.ShapeDtypeStruct((N, D), F32)] * 2,
    )(v, w1a, w1b, b1)


def _gather_body(va_hbm, vb_hbm, src_hbm, dst_hbm,
                 sr_out,
                 isrc_v, idst_v, a0, a1, a2, b0, b1, b2,
                 sga0, sga1, sga2, sgb0, sgb1, sgb2, swa0, swa1, swa2):
    c = lax.axis_index("c")
    s = lax.axis_index("s")
    wid = s * NC + c
    wbase = wid * MW

    pltpu.sync_copy(src_hbm.at[wid], isrc_v)
    pltpu.sync_copy(dst_hbm.at[wid], idst_v)

    def isl(iv, j):
        return iv.at[j]

    def osl(j):
        return sr_out.at[pl.ds(wbase + j * CW, CW)]

    def vadd(dst_ref, src_ref):
        def rbody(r, carry):
            for k in range(D // 16):
                sl = pl.ds(k * 16, 16)
                dst_ref[r, sl] = dst_ref[r, sl] + src_ref[r, sl]
            return carry
        lax.fori_loop(0, CW, rbody, 0)

    abufs = (a0, a1, a2)
    bbufs = (b0, b1, b2)
    sgas = (sga0, sga1, sga2)
    sgbs = (sgb0, sgb1, sgb2)
    swas = (swa0, swa1, swa2)
    pltpu.async_copy(va_hbm.at[isl(isrc_v, 0)], a0, sga0)
    pltpu.async_copy(vb_hbm.at[isl(idst_v, 0)], b0, sgb0)
    pltpu.async_copy(va_hbm.at[isl(isrc_v, 1)], a1, sga1)
    pltpu.async_copy(vb_hbm.at[isl(idst_v, 1)], b1, sgb1)

    def step(j, p):
        cur_a, cur_b = abufs[p], bbufs[p]
        n2 = (p + 2) % 3
        nxt_a, nxt_b = abufs[n2], bbufs[n2]
        pltpu.make_async_copy(va_hbm.at[isl(isrc_v, j)], cur_a, sgas[p]).wait()
        pltpu.make_async_copy(vb_hbm.at[isl(idst_v, j)], cur_b, sgbs[p]).wait()

        @pl.when(j + 2 < CHUNKS)
        def _():
            @pl.when(j >= 1)
            def _():
                pltpu.make_async_copy(nxt_a, osl(j - 1), swas[n2]).wait()
            pltpu.async_copy(va_hbm.at[isl(isrc_v, j + 2)], nxt_a, sgas[n2])
            pltpu.async_copy(vb_hbm.at[isl(idst_v, j + 2)], nxt_b, sgbs[n2])

        vadd(cur_a, cur_b)
        pltpu.async_copy(cur_a, osl(j), swas[p])

    def body(j, carry):
        for p in range(3):
            @pl.when(j % 3 == p)
            def _(p=p):
                step(j, p)
        return carry

    lax.fori_loop(0, CHUNKS, body, 0)
    for t in (CHUNKS - 3, CHUNKS - 2, CHUNKS - 1):
        pltpu.make_async_copy(abufs[t % 3], osl(t), swas[t % 3]).wait()


_gather = functools.partial(
    pl.kernel,
    out_type=jax.ShapeDtypeStruct((M, D), F32),
    mesh=_mesh,
    scratch_types=[
        pltpu.VMEM((CHUNKS, CW), jnp.int32),
        pltpu.VMEM((CHUNKS, CW), jnp.int32),
        pltpu.VMEM((CW, D), F32),
        pltpu.VMEM((CW, D), F32),
        pltpu.VMEM((CW, D), F32),
        pltpu.VMEM((CW, D), F32),
        pltpu.VMEM((CW, D), F32),
        pltpu.VMEM((CW, D), F32),
    ] + [pltpu.SemaphoreType.DMA] * 9,
)(_gather_body)


def _counts_body(dst_hbm, zeros_h, ones_h, pcnt_out,
                 idx_v, ones_v, cnt_sh):
    c = lax.axis_index("c")
    s = lax.axis_index("s")
    wid = s * NC + c

    pltpu.sync_copy(zeros_h.at[pl.ds(s * RPT, RPT)],
                    cnt_sh.at[pl.ds(s * RPT, RPT)])
    pltpu.sync_copy(ones_h, ones_v)
    pltpu.sync_copy(dst_hbm.at[wid], idx_v)
    plsc.subcore_barrier()

    def body(j, carry):
        pltpu.sync_copy(ones_v, cnt_sh.at[idx_v.at[j]], add=True)
        return carry

    lax.fori_loop(0, CHUNKS, body, 0)
    plsc.subcore_barrier()
    pltpu.sync_copy(cnt_sh.at[pl.ds(s * RPT, RPT)],
                    pcnt_out.at[c, pl.ds(s * RPT, RPT)])


_counts = functools.partial(
    pl.kernel,
    out_type=jax.ShapeDtypeStruct((NC, NP, D), F32),
    mesh=_mesh,
    scratch_types=[
        pltpu.VMEM((CHUNKS, CW), jnp.int32),
        pltpu.VMEM((CW, D), F32),
        pltpu.VMEM_SHARED((NP, D), F32),
    ],
)(_counts_body)


def _scatter_body(eemb, dstidx, zeros_h, psum,
                  idx_v, r0, r1, r2, acc_sh, sr0, sr1, sr2):
    c = lax.axis_index("c")
    s = lax.axis_index("s")
    wid = s * NC + c
    wbase = wid * MW

    pltpu.sync_copy(zeros_h.at[pl.ds(s * RPT, RPT)],
                    acc_sh.at[pl.ds(s * RPT, RPT)])
    pltpu.sync_copy(dstidx.at[wid], idx_v)
    plsc.subcore_barrier()

    def esl(j):
        return eemb.at[pl.ds(wbase + j * CW, CW)]

    pltpu.async_copy(esl(0), r0, sr0)
    pltpu.async_copy(esl(1), r1, sr1)

    def step(j, cur, nxt2, scur, snxt2):
        pltpu.make_async_copy(esl(j), cur, scur).wait()

        @pl.when(j + 2 < CHUNKS)
        def _():
            pltpu.async_copy(esl(j + 2), nxt2, snxt2)

        pltpu.sync_copy(cur, acc_sh.at[idx_v.at[j]], add=True)

    def body(j, carry):
        @pl.when(j % 3 == 0)
        def _():
            step(j, r0, r2, sr0, sr2)

        @pl.when(j % 3 == 1)
        def _():
            step(j, r1, r0, sr1, sr0)

        @pl.when(j % 3 == 2)
        def _():
            step(j, r2, r1, sr2, sr1)
        return carry

    lax.fori_loop(0, CHUNKS, body, 0)
    plsc.subcore_barrier()
    pltpu.sync_copy(acc_sh.at[pl.ds(s * RPT, RPT)],
                    psum.at[c, pl.ds(s * RPT, RPT)])


_scatter = functools.partial(
    pl.kernel,
    out_type=jax.ShapeDtypeStruct((NC, NP, D), F32),
    mesh=_mesh,
    scratch_types=[
        pltpu.VMEM((CHUNKS, CW), jnp.int32),
        pltpu.VMEM((CW, D), F32),
        pltpu.VMEM((CW, D), F32),
        pltpu.VMEM((CW, D), F32),
        pltpu.VMEM_SHARED((NP, D), F32),
        pltpu.SemaphoreType.DMA,
        pltpu.SemaphoreType.DMA,
        pltpu.SemaphoreType.DMA,
    ],
)(_scatter_body)


def _edge_body(sr_ref, e_ref, w1c_ref, w2_ref, b2_ref, w3_ref, b3_ref,
               out_ref):
    x = sr_ref[...] + jnp.dot(
        e_ref[...], w1c_ref[...], preferred_element_type=F32)
    h1 = jnp.maximum(x, 0.0)
    h2 = jnp.maximum(
        jnp.dot(h1, w2_ref[...], preferred_element_type=F32) + b2_ref[...], 0.0)
    out_ref[...] = jnp.dot(h2, w3_ref[...], preferred_element_type=F32) + b3_ref[...]


def _edge(sr, e, w1c, w2, b2, w3, b3):
    bm = 8000
    grid = M // bm
    wspec = pl.BlockSpec((D, D), lambda j: (0, 0))
    bspec = pl.BlockSpec((1, D), lambda j: (0, 0))
    xspec = pl.BlockSpec((bm, D), lambda j: (j, 0))
    return pl.pallas_call(
        _edge_body,
        grid=(grid,),
        in_specs=[xspec, xspec, wspec, wspec, bspec, wspec, bspec],
        out_specs=xspec,
        out_shape=jax.ShapeDtypeStruct((M, D), F32),
        compiler_params=pltpu.CompilerParams(
            dimension_semantics=("arbitrary",)),
    )(sr, e, w1c, w2, b2, w3, b3)


def _node_body(v_ref, ps_ref, pc_ref, w1a_ref, w1b_ref, b1_ref, w2_ref,
               b2_ref, w3_ref, b3_ref, out_ref):
    ssum = ps_ref[0] + ps_ref[1]
    cnt = pc_ref[0, :, 0:1] + pc_ref[1, :, 0:1]
    mean = ssum / jnp.maximum(cnt, 1.0)
    x = (jnp.dot(v_ref[...], w1a_ref[...], preferred_element_type=F32)
         + jnp.dot(mean, w1b_ref[...], preferred_element_type=F32)
         + b1_ref[...])
    h1 = jnp.maximum(x, 0.0)
    h2 = jnp.maximum(
        jnp.dot(h1, w2_ref[...], preferred_element_type=F32) + b2_ref[...], 0.0)
    out_ref[...] = jnp.dot(h2, w3_ref[...], preferred_element_type=F32) + b3_ref[...]


def _node(v, psum, pcnt, w1a, w1b, b1, w2, b2, w3, b3):
    bn = 1000
    grid = N // bn
    wspec = pl.BlockSpec((D, D), lambda j: (0, 0))
    bspec = pl.BlockSpec((1, D), lambda j: (0, 0))
    return pl.pallas_call(
        _node_body,
        grid=(grid,),
        in_specs=[
            pl.BlockSpec((bn, D), lambda j: (j, 0)),
            pl.BlockSpec((NC, bn, D), lambda j: (0, j, 0)),
            pl.BlockSpec((NC, bn, D), lambda j: (0, j, 0)),
            wspec, wspec, bspec, wspec, bspec, wspec, bspec,
        ],
        out_specs=pl.BlockSpec((bn, D), lambda j: (j, 0)),
        out_shape=jax.ShapeDtypeStruct((N, D), F32),
    )(v, psum, pcnt, w1a, w1b, b1, w2, b2, w3, b3)


def kernel(V, E, edges, fe_W1, fe_b1, fe_W2, fe_b2, fe_W3, fe_b3,
           fn_W1, fn_b1, fn_W2, fn_b2, fn_W3, fn_b3):
    v2 = V[0]
    e2 = E[0]
    edges = edges.astype(jnp.int32)
    src = edges[0, :, 0].reshape(NW, CHUNKS, CW)
    dst = edges[0, :, 1].reshape(NW, CHUNKS, CW)

    zeros_h = jnp.zeros((NP, D), F32)
    ones_h = jnp.ones((CW, D), F32)

    va, vb = _prep(v2, fe_W1[:D], fe_W1[D:2 * D], fe_b1.reshape(1, D))
    sr_rows = _gather(va, vb, src, dst)
    pcnt = _counts(dst, zeros_h, ones_h)
    eemb = _edge(sr_rows, e2, fe_W1[2 * D:], fe_W2,
                 fe_b2.reshape(1, D), fe_W3, fe_b3.reshape(1, D))

    psum = _scatter(eemb, dst, zeros_h)

    nemb = _node(v2, psum[:, :N], pcnt[:, :N],
                 fn_W1[:D], fn_W1[D:], fn_b1.reshape(1, D),
                 fn_W2, fn_b2.reshape(1, D), fn_W3, fn_b3.reshape(1, D))
    return (nemb[None], eemb[None])


# speedup vs baseline: 1.0143x; 1.0143x over previous
import functools

import jax
import jax.numpy as jnp
from jax import lax
from jax.experimental import pallas as pl
from jax.experimental.pallas import tpu as pltpu
from jax.experimental.pallas import tpu_sc as plsc

N = 10000
M = 320000
D = 128
NC = 2
NS = 16
NW = NC * NS
MW = M // NW
CW = 80
CHUNKS = MW // CW
NP = 10240
RPT = NP // NS

_mesh = plsc.VectorSubcoreMesh(
    core_axis_name="c", subcore_axis_name="s", num_cores=NC, num_subcores=NS)

F32 = jnp.float32


def _prep_body(v_ref, w1a_ref, w1b_ref, b1_ref, va_ref, vb_ref):
    v = v_ref[...]
    va_ref[...] = jnp.dot(v, w1a_ref[...], preferred_element_type=F32)
    vb_ref[...] = jnp.dot(v, w1b_ref[...], preferred_element_type=F32) + b1_ref[...]


def _prep(v, w1a, w1b, b1):
    bn = 1000
    grid = N // bn
    return pl.pallas_call(
        _prep_body,
        grid=(grid,),
        in_specs=[
            pl.BlockSpec((bn, D), lambda j: (j, 0)),
            pl.BlockSpec((D, D), lambda j: (0, 0)),
            pl.BlockSpec((D, D), lambda j: (0, 0)),
            pl.BlockSpec((1, D), lambda j: (0, 0)),
        ],
        out_specs=[
            pl.BlockSpec((bn, D), lambda j: (j, 0)),
            pl.BlockSpec((bn, D), lambda j: (j, 0)),
        ],
        out_shape=[jax.ShapeDtypeStruct((N, D), F32)] * 2,
    )(v, w1a, w1b, b1)


def _gather_body(va_hbm, vb_hbm, src_hbm, dst_hbm,
                 sr_out,
                 isrc_v, idst_v, a0, a1, a2, b0, b1, b2,
                 sga0, sga1, sga2, sgb0, sgb1, sgb2, swa0, swa1, swa2):
    c = lax.axis_index("c")
    s = lax.axis_index("s")
    wid = s * NC + c
    wbase = wid * MW

    pltpu.sync_copy(src_hbm.at[wid], isrc_v)
    pltpu.sync_copy(dst_hbm.at[wid], idst_v)

    def isl(iv, j):
        return iv.at[j]

    def osl(j):
        return sr_out.at[pl.ds(wbase + j * CW, CW)]

    def vadd(dst_ref, src_ref):
        def rbody(r, carry):
            for k in range(D // 16):
                sl = pl.ds(k * 16, 16)
                dst_ref[r, sl] = dst_ref[r, sl] + src_ref[r, sl]
            return carry
        lax.fori_loop(0, CW, rbody, 0)

    abufs = (a0, a1, a2)
    bbufs = (b0, b1, b2)
    sgas = (sga0, sga1, sga2)
    sgbs = (sgb0, sgb1, sgb2)
    swas = (swa0, swa1, swa2)
    pltpu.async_copy(va_hbm.at[isl(isrc_v, 0)], a0, sga0)
    pltpu.async_copy(vb_hbm.at[isl(idst_v, 0)], b0, sgb0)
    pltpu.async_copy(va_hbm.at[isl(isrc_v, 1)], a1, sga1)
    pltpu.async_copy(vb_hbm.at[isl(idst_v, 1)], b1, sgb1)

    def step(j, p):
        cur_a, cur_b = abufs[p], bbufs[p]
        n2 = (p + 2) % 3
        nxt_a, nxt_b = abufs[n2], bbufs[n2]
        pltpu.make_async_copy(va_hbm.at[isl(isrc_v, j)], cur_a, sgas[p]).wait()
        pltpu.make_async_copy(vb_hbm.at[isl(idst_v, j)], cur_b, sgbs[p]).wait()

        @pl.when(j + 2 < CHUNKS)
        def _():
            @pl.when(j >= 1)
            def _():
                pltpu.make_async_copy(nxt_a, osl(j - 1), swas[n2]).wait()
            pltpu.async_copy(va_hbm.at[isl(isrc_v, j + 2)], nxt_a, sgas[n2])
            pltpu.async_copy(vb_hbm.at[isl(idst_v, j + 2)], nxt_b, sgbs[n2])

        vadd(cur_a, cur_b)
        pltpu.async_copy(cur_a, osl(j), swas[p])

    def body(j, carry):
        for p in range(3):
            @pl.when(j % 3 == p)
            def _(p=p):
                step(j, p)
        return carry

    lax.fori_loop(0, CHUNKS, body, 0)
    for t in (CHUNKS - 3, CHUNKS - 2, CHUNKS - 1):
        pltpu.make_async_copy(abufs[t % 3], osl(t), swas[t % 3]).wait()


_gather = functools.partial(
    pl.kernel,
    out_type=jax.ShapeDtypeStruct((M, D), F32),
    mesh=_mesh,
    scratch_types=[
        pltpu.VMEM((CHUNKS, CW), jnp.int32),
        pltpu.VMEM((CHUNKS, CW), jnp.int32),
        pltpu.VMEM((CW, D), F32),
        pltpu.VMEM((CW, D), F32),
        pltpu.VMEM((CW, D), F32),
        pltpu.VMEM((CW, D), F32),
        pltpu.VMEM((CW, D), F32),
        pltpu.VMEM((CW, D), F32),
    ] + [pltpu.SemaphoreType.DMA] * 9,
)(_gather_body)


def _counts_body(dst_hbm, zeros_h, ones_h, pcnt_out,
                 idx_v, ones_v, cnt_sh, sa0, sa1):
    c = lax.axis_index("c")
    s = lax.axis_index("s")
    wid = s * NC + c

    pltpu.sync_copy(zeros_h.at[pl.ds(s * RPT, RPT)],
                    cnt_sh.at[pl.ds(s * RPT, RPT)])
    pltpu.sync_copy(ones_h, ones_v)
    pltpu.sync_copy(dst_hbm.at[wid], idx_v)
    plsc.subcore_barrier()

    sems = (sa0, sa1)

    def step(j, p):
        @pl.when(j >= 2)
        def _():
            pltpu.make_async_copy(
                ones_v, cnt_sh.at[idx_v.at[j - 2]], sems[p]).wait()
        pltpu.async_copy(ones_v, cnt_sh.at[idx_v.at[j]], sems[p], add=True)

    def body(j, carry):
        for p in range(2):
            @pl.when(j % 2 == p)
            def _(p=p):
                step(j, p)
        return carry

    lax.fori_loop(0, CHUNKS, body, 0)
    for t in (CHUNKS - 2, CHUNKS - 1):
        pltpu.make_async_copy(
            ones_v, cnt_sh.at[idx_v.at[t]], sems[t % 2]).wait()
    plsc.subcore_barrier()
    pltpu.sync_copy(cnt_sh.at[pl.ds(s * RPT, RPT)],
                    pcnt_out.at[c, pl.ds(s * RPT, RPT)])


_counts = functools.partial(
    pl.kernel,
    out_type=jax.ShapeDtypeStruct((NC, NP, D), F32),
    mesh=_mesh,
    scratch_types=[
        pltpu.VMEM((CHUNKS, CW), jnp.int32),
        pltpu.VMEM((CW, D), F32),
        pltpu.VMEM_SHARED((NP, D), F32),
        pltpu.SemaphoreType.DMA,
        pltpu.SemaphoreType.DMA,
    ],
)(_counts_body)


def _scatter_body(eemb, dstidx, zeros_h, psum,
                  idx_v, r0, r1, r2, acc_sh, sr0, sr1, sr2):
    c = lax.axis_index("c")
    s = lax.axis_index("s")
    wid = s * NC + c
    wbase = wid * MW

    pltpu.sync_copy(zeros_h.at[pl.ds(s * RPT, RPT)],
                    acc_sh.at[pl.ds(s * RPT, RPT)])
    pltpu.sync_copy(dstidx.at[wid], idx_v)
    plsc.subcore_barrier()

    def esl(j):
        return eemb.at[pl.ds(wbase + j * CW, CW)]

    pltpu.async_copy(esl(0), r0, sr0)
    pltpu.async_copy(esl(1), r1, sr1)

    def step(j, cur, nxt2, scur, snxt2):
        pltpu.make_async_copy(esl(j), cur, scur).wait()

        @pl.when(j + 2 < CHUNKS)
        def _():
            pltpu.async_copy(esl(j + 2), nxt2, snxt2)

        pltpu.sync_copy(cur, acc_sh.at[idx_v.at[j]], add=True)

    def body(j, carry):
        @pl.when(j % 3 == 0)
        def _():
            step(j, r0, r2, sr0, sr2)

        @pl.when(j % 3 == 1)
        def _():
            step(j, r1, r0, sr1, sr0)

        @pl.when(j % 3 == 2)
        def _():
            step(j, r2, r1, sr2, sr1)
        return carry

    lax.fori_loop(0, CHUNKS, body, 0)
    plsc.subcore_barrier()
    pltpu.sync_copy(acc_sh.at[pl.ds(s * RPT, RPT)],
                    psum.at[c, pl.ds(s * RPT, RPT)])


_scatter = functools.partial(
    pl.kernel,
    out_type=jax.ShapeDtypeStruct((NC, NP, D), F32),
    mesh=_mesh,
    scratch_types=[
        pltpu.VMEM((CHUNKS, CW), jnp.int32),
        pltpu.VMEM((CW, D), F32),
        pltpu.VMEM((CW, D), F32),
        pltpu.VMEM((CW, D), F32),
        pltpu.VMEM_SHARED((NP, D), F32),
        pltpu.SemaphoreType.DMA,
        pltpu.SemaphoreType.DMA,
        pltpu.SemaphoreType.DMA,
    ],
)(_scatter_body)


def _edge_body(sr_ref, e_ref, w1c_ref, w2_ref, b2_ref, w3_ref, b3_ref,
               out_ref):
    x = sr_ref[...] + jnp.dot(
        e_ref[...], w1c_ref[...], preferred_element_type=F32)
    h1 = jnp.maximum(x, 0.0)
    h2 = jnp.maximum(
        jnp.dot(h1, w2_ref[...], preferred_element_type=F32) + b2_ref[...], 0.0)
    out_ref[...] = jnp.dot(h2, w3_ref[...], preferred_element_type=F32) + b3_ref[...]


def _edge(sr, e, w1c, w2, b2, w3, b3):
    bm = 16000
    grid = M // bm
    wspec = pl.BlockSpec((D, D), lambda j: (0, 0))
    bspec = pl.BlockSpec((1, D), lambda j: (0, 0))
    xspec = pl.BlockSpec((bm, D), lambda j: (j, 0))
    return pl.pallas_call(
        _edge_body,
        grid=(grid,),
        in_specs=[xspec, xspec, wspec, wspec, bspec, wspec, bspec],
        out_specs=xspec,
        out_shape=jax.ShapeDtypeStruct((M, D), F32),
        compiler_params=pltpu.CompilerParams(
            dimension_semantics=("arbitrary",)),
    )(sr, e, w1c, w2, b2, w3, b3)


def _node_body(v_ref, ps_ref, pc_ref, w1a_ref, w1b_ref, b1_ref, w2_ref,
               b2_ref, w3_ref, b3_ref, out_ref):
    ssum = ps_ref[0] + ps_ref[1]
    cnt = pc_ref[0, :, 0:1] + pc_ref[1, :, 0:1]
    mean = ssum / jnp.maximum(cnt, 1.0)
    x = (jnp.dot(v_ref[...], w1a_ref[...], preferred_element_type=F32)
         + jnp.dot(mean, w1b_ref[...], preferred_element_type=F32)
         + b1_ref[...])
    h1 = jnp.maximum(x, 0.0)
    h2 = jnp.maximum(
        jnp.dot(h1, w2_ref[...], preferred_element_type=F32) + b2_ref[...], 0.0)
    out_ref[...] = jnp.dot(h2, w3_ref[...], preferred_element_type=F32) + b3_ref[...]


def _node(v, psum, pcnt, w1a, w1b, b1, w2, b2, w3, b3):
    bn = 1000
    grid = N // bn
    wspec = pl.BlockSpec((D, D), lambda j: (0, 0))
    bspec = pl.BlockSpec((1, D), lambda j: (0, 0))
    return pl.pallas_call(
        _node_body,
        grid=(grid,),
        in_specs=[
            pl.BlockSpec((bn, D), lambda j: (j, 0)),
            pl.BlockSpec((NC, bn, D), lambda j: (0, j, 0)),
            pl.BlockSpec((NC, bn, D), lambda j: (0, j, 0)),
            wspec, wspec, bspec, wspec, bspec, wspec, bspec,
        ],
        out_specs=pl.BlockSpec((bn, D), lambda j: (j, 0)),
        out_shape=jax.ShapeDtypeStruct((N, D), F32),
    )(v, psum, pcnt, w1a, w1b, b1, w2, b2, w3, b3)


def kernel(V, E, edges, fe_W1, fe_b1, fe_W2, fe_b2, fe_W3, fe_b3,
           fn_W1, fn_b1, fn_W2, fn_b2, fn_W3, fn_b3):
    v2 = V[0]
    e2 = E[0]
    edges = edges.astype(jnp.int32)
    src = edges[0, :, 0].reshape(NW, CHUNKS, CW)
    dst = edges[0, :, 1].reshape(NW, CHUNKS, CW)

    zeros_h = jnp.zeros((NP, D), F32)
    ones_h = jnp.ones((CW, D), F32)

    va, vb = _prep(v2, fe_W1[:D], fe_W1[D:2 * D], fe_b1.reshape(1, D))
    sr_rows = _gather(va, vb, src, dst)
    pcnt = _counts(dst, zeros_h, ones_h)
    eemb = _edge(sr_rows, e2, fe_W1[2 * D:], fe_W2,
                 fe_b2.reshape(1, D), fe_W3, fe_b3.reshape(1, D))

    psum = _scatter(eemb, dst, zeros_h)

    nemb = _node(v2, psum[:, :N], pcnt[:, :N],
                 fn_W1[:D], fn_W1[D:], fn_b1.reshape(1, D),
                 fn_W2, fn_b2.reshape(1, D), fn_W3, fn_b3.reshape(1, D))
    return (nemb[None], eemb[None])


# speedup vs baseline: 1.0158x; 1.0015x over previous
import functools

import jax
import jax.numpy as jnp
from jax import lax
from jax.experimental import pallas as pl
from jax.experimental.pallas import tpu as pltpu
from jax.experimental.pallas import tpu_sc as plsc

N = 10000
M = 320000
D = 128
NC = 2
NS = 16
NW = NC * NS
MW = M // NW
CW = 80
CHUNKS = MW // CW
NP = 10240
RPT = NP // NS

_mesh = plsc.VectorSubcoreMesh(
    core_axis_name="c", subcore_axis_name="s", num_cores=NC, num_subcores=NS)

F32 = jnp.float32


def _prep_body(v_ref, w1a_ref, w1b_ref, b1_ref, va_ref, vb_ref):
    v = v_ref[...]
    va_ref[...] = jnp.dot(v, w1a_ref[...], preferred_element_type=F32)
    vb_ref[...] = jnp.dot(v, w1b_ref[...], preferred_element_type=F32) + b1_ref[...]


def _prep(v, w1a, w1b, b1):
    bn = 1000
    grid = N // bn
    return pl.pallas_call(
        _prep_body,
        grid=(grid,),
        in_specs=[
            pl.BlockSpec((bn, D), lambda j: (j, 0)),
            pl.BlockSpec((D, D), lambda j: (0, 0)),
            pl.BlockSpec((D, D), lambda j: (0, 0)),
            pl.BlockSpec((1, D), lambda j: (0, 0)),
        ],
        out_specs=[
            pl.BlockSpec((bn, D), lambda j: (j, 0)),
            pl.BlockSpec((bn, D), lambda j: (j, 0)),
        ],
        out_shape=[jax.ShapeDtypeStruct((N, D), F32)] * 2,
    )(v, w1a, w1b, b1)


def _gather_body(va_hbm, vb_hbm, src_hbm, dst_hbm,
                 sr_out,
                 isrc_v, idst_v, a0, a1, a2, b0, b1, b2,
                 sga0, sga1, sga2, sgb0, sgb1, sgb2, swa0, swa1, swa2):
    c = lax.axis_index("c")
    s = lax.axis_index("s")
    wid = s * NC + c
    wbase = wid * MW

    pltpu.sync_copy(src_hbm.at[wid], isrc_v)
    pltpu.sync_copy(dst_hbm.at[wid], idst_v)

    def isl(iv, j):
        return iv.at[j]

    def osl(j):
        return sr_out.at[pl.ds(wbase + j * CW, CW)]

    def vadd(dst_ref, src_ref):
        def rbody(r, carry):
            for k in range(D // 16):
                sl = pl.ds(k * 16, 16)
                dst_ref[r, sl] = dst_ref[r, sl] + src_ref[r, sl]
            return carry
        lax.fori_loop(0, CW, rbody, 0)

    abufs = (a0, a1, a2)
    bbufs = (b0, b1, b2)
    sgas = (sga0, sga1, sga2)
    sgbs = (sgb0, sgb1, sgb2)
    swas = (swa0, swa1, swa2)
    pltpu.async_copy(va_hbm.at[isl(isrc_v, 0)], a0, sga0)
    pltpu.async_copy(vb_hbm.at[isl(idst_v, 0)], b0, sgb0)
    pltpu.async_copy(va_hbm.at[isl(isrc_v, 1)], a1, sga1)
    pltpu.async_copy(vb_hbm.at[isl(idst_v, 1)], b1, sgb1)

    def step(j, p):
        cur_a, cur_b = abufs[p], bbufs[p]
        n2 = (p + 2) % 3
        nxt_a, nxt_b = abufs[n2], bbufs[n2]
        pltpu.make_async_copy(va_hbm.at[isl(isrc_v, j)], cur_a, sgas[p]).wait()
        pltpu.make_async_copy(vb_hbm.at[isl(idst_v, j)], cur_b, sgbs[p]).wait()

        @pl.when(j + 2 < CHUNKS)
        def _():
            @pl.when(j >= 1)
            def _():
                pltpu.make_async_copy(nxt_a, osl(j - 1), swas[n2]).wait()
            pltpu.async_copy(va_hbm.at[isl(isrc_v, j + 2)], nxt_a, sgas[n2])
            pltpu.async_copy(vb_hbm.at[isl(idst_v, j + 2)], nxt_b, sgbs[n2])

        vadd(cur_a, cur_b)
        pltpu.async_copy(cur_a, osl(j), swas[p])

    def body(j, carry):
        for p in range(3):
            @pl.when(j % 3 == p)
            def _(p=p):
                step(j, p)
        return carry

    lax.fori_loop(0, CHUNKS, body, 0)
    for t in (CHUNKS - 3, CHUNKS - 2, CHUNKS - 1):
        pltpu.make_async_copy(abufs[t % 3], osl(t), swas[t % 3]).wait()


_gather = functools.partial(
    pl.kernel,
    out_type=jax.ShapeDtypeStruct((M, D), F32),
    mesh=_mesh,
    scratch_types=[
        pltpu.VMEM((CHUNKS, CW), jnp.int32),
        pltpu.VMEM((CHUNKS, CW), jnp.int32),
        pltpu.VMEM((CW, D), F32),
        pltpu.VMEM((CW, D), F32),
        pltpu.VMEM((CW, D), F32),
        pltpu.VMEM((CW, D), F32),
        pltpu.VMEM((CW, D), F32),
        pltpu.VMEM((CW, D), F32),
    ] + [pltpu.SemaphoreType.DMA] * 9,
)(_gather_body)


def _counts_body(dst_hbm, zeros_h, ones_h, pcnt_out,
                 idx_v, ones_v, cnt_sh):
    c = lax.axis_index("c")
    s = lax.axis_index("s")
    wid = s * NC + c

    pltpu.sync_copy(zeros_h.at[pl.ds(s * RPT, RPT)],
                    cnt_sh.at[pl.ds(s * RPT, RPT)])
    pltpu.sync_copy(ones_h, ones_v)
    pltpu.sync_copy(dst_hbm.at[wid], idx_v)
    plsc.subcore_barrier()

    def body(j, carry):
        pltpu.sync_copy(ones_v, cnt_sh.at[idx_v.at[j]], add=True)
        return carry

    lax.fori_loop(0, CHUNKS, body, 0)
    plsc.subcore_barrier()
    pltpu.sync_copy(cnt_sh.at[pl.ds(s * RPT, RPT)],
                    pcnt_out.at[c, pl.ds(s * RPT, RPT)])


_counts = functools.partial(
    pl.kernel,
    out_type=jax.ShapeDtypeStruct((NC, NP, D), F32),
    mesh=_mesh,
    scratch_types=[
        pltpu.VMEM((CHUNKS, CW), jnp.int32),
        pltpu.VMEM((CW, D), F32),
        pltpu.VMEM_SHARED((NP, D), F32),
    ],
)(_counts_body)


def _scatter_body(eemb, dstidx, zeros_h, psum,
                  idx_v, r0, r1, r2, acc_sh, sr0, sr1, sr2):
    c = lax.axis_index("c")
    s = lax.axis_index("s")
    wid = s * NC + c
    wbase = wid * MW

    pltpu.sync_copy(zeros_h.at[pl.ds(s * RPT, RPT)],
                    acc_sh.at[pl.ds(s * RPT, RPT)])
    pltpu.sync_copy(dstidx.at[wid], idx_v)
    plsc.subcore_barrier()

    def esl(j):
        return eemb.at[pl.ds(wbase + j * CW, CW)]

    pltpu.async_copy(esl(0), r0, sr0)
    pltpu.async_copy(esl(1), r1, sr1)

    def step(j, cur, nxt2, scur, snxt2):
        pltpu.make_async_copy(esl(j), cur, scur).wait()

        @pl.when(j + 2 < CHUNKS)
        def _():
            pltpu.async_copy(esl(j + 2), nxt2, snxt2)

        pltpu.sync_copy(cur, acc_sh.at[idx_v.at[j]], add=True)

    def body(j, carry):
        @pl.when(j % 3 == 0)
        def _():
            step(j, r0, r2, sr0, sr2)

        @pl.when(j % 3 == 1)
        def _():
            step(j, r1, r0, sr1, sr0)

        @pl.when(j % 3 == 2)
        def _():
            step(j, r2, r1, sr2, sr1)
        return carry

    lax.fori_loop(0, CHUNKS, body, 0)
    plsc.subcore_barrier()
    pltpu.sync_copy(acc_sh.at[pl.ds(s * RPT, RPT)],
                    psum.at[c, pl.ds(s * RPT, RPT)])


_scatter = functools.partial(
    pl.kernel,
    out_type=jax.ShapeDtypeStruct((NC, NP, D), F32),
    mesh=_mesh,
    scratch_types=[
        pltpu.VMEM((CHUNKS, CW), jnp.int32),
        pltpu.VMEM((CW, D), F32),
        pltpu.VMEM((CW, D), F32),
        pltpu.VMEM((CW, D), F32),
        pltpu.VMEM_SHARED((NP, D), F32),
        pltpu.SemaphoreType.DMA,
        pltpu.SemaphoreType.DMA,
        pltpu.SemaphoreType.DMA,
    ],
)(_scatter_body)


def _edge_body(sr_ref, e_ref, w1c_ref, w2_ref, b2_ref, w3_ref, b3_ref,
               out_ref):
    x = sr_ref[...] + jnp.dot(
        e_ref[...], w1c_ref[...], preferred_element_type=F32)
    h1 = jnp.maximum(x, 0.0)
    h2 = jnp.maximum(
        jnp.dot(h1, w2_ref[...], preferred_element_type=F32) + b2_ref[...], 0.0)
    out_ref[...] = jnp.dot(h2, w3_ref[...], preferred_element_type=F32) + b3_ref[...]


def _edge(sr, e, w1c, w2, b2, w3, b3):
    bm = 16000
    grid = M // bm
    wspec = pl.BlockSpec((D, D), lambda j: (0, 0))
    bspec = pl.BlockSpec((1, D), lambda j: (0, 0))
    xspec = pl.BlockSpec((bm, D), lambda j: (j, 0))
    return pl.pallas_call(
        _edge_body,
        grid=(grid,),
        in_specs=[xspec, xspec, wspec, wspec, bspec, wspec, bspec],
        out_specs=xspec,
        out_shape=jax.ShapeDtypeStruct((M, D), F32),
        compiler_params=pltpu.CompilerParams(
            dimension_semantics=("arbitrary",)),
    )(sr, e, w1c, w2, b2, w3, b3)


def _node_body(v_ref, ps_ref, pc_ref, w1a_ref, w1b_ref, b1_ref, w2_ref,
               b2_ref, w3_ref, b3_ref, out_ref):
    ssum = ps_ref[0] + ps_ref[1]
    cnt = pc_ref[0, :, 0:1] + pc_ref[1, :, 0:1]
    mean = ssum / jnp.maximum(cnt, 1.0)
    x = (jnp.dot(v_ref[...], w1a_ref[...], preferred_element_type=F32)
         + jnp.dot(mean, w1b_ref[...], preferred_element_type=F32)
         + b1_ref[...])
    h1 = jnp.maximum(x, 0.0)
    h2 = jnp.maximum(
        jnp.dot(h1, w2_ref[...], preferred_element_type=F32) + b2_ref[...], 0.0)
    out_ref[...] = jnp.dot(h2, w3_ref[...], preferred_element_type=F32) + b3_ref[...]


def _node(v, psum, pcnt, w1a, w1b, b1, w2, b2, w3, b3):
    bn = 1000
    grid = N // bn
    wspec = pl.BlockSpec((D, D), lambda j: (0, 0))
    bspec = pl.BlockSpec((1, D), lambda j: (0, 0))
    return pl.pallas_call(
        _node_body,
        grid=(grid,),
        in_specs=[
            pl.BlockSpec((bn, D), lambda j: (j, 0)),
            pl.BlockSpec((NC, bn, D), lambda j: (0, j, 0)),
            pl.BlockSpec((NC, bn, D), lambda j: (0, j, 0)),
            wspec, wspec, bspec, wspec, bspec, wspec, bspec,
        ],
        out_specs=pl.BlockSpec((bn, D), lambda j: (j, 0)),
        out_shape=jax.ShapeDtypeStruct((N, D), F32),
    )(v, psum, pcnt, w1a, w1b, b1, w2, b2, w3, b3)


def kernel(V, E, edges, fe_W1, fe_b1, fe_W2, fe_b2, fe_W3, fe_b3,
           fn_W1, fn_b1, fn_W2, fn_b2, fn_W3, fn_b3):
    v2 = V[0]
    e2 = E[0]
    edges = edges.astype(jnp.int32)
    src = edges[0, :, 0].reshape(NW, CHUNKS, CW)
    dst = edges[0, :, 1].reshape(NW, CHUNKS, CW)

    zeros_h = jnp.zeros((NP, D), F32)
    ones_h = jnp.ones((CW, D), F32)

    va, vb = _prep(v2, fe_W1[:D], fe_W1[D:2 * D], fe_b1.reshape(1, D))
    sr_rows = _gather(va, vb, src, dst)
    pcnt = _counts(dst, zeros_h, ones_h)
    eemb = _edge(sr_rows, e2, fe_W1[2 * D:], fe_W2,
                 fe_b2.reshape(1, D), fe_W3, fe_b3.reshape(1, D))

    psum = _scatter(eemb, dst, zeros_h)

    nemb = _node(v2, psum[:, :N], pcnt[:, :N],
                 fn_W1[:D], fn_W1[D:], fn_b1.reshape(1, D),
                 fn_W2, fn_b2.reshape(1, D), fn_W3, fn_b3.reshape(1, D))
    return (nemb[None], eemb[None])


# speedup vs baseline: 1.0260x; 1.0100x over previous
import functools

import jax
import jax.numpy as jnp
from jax import lax
from jax.experimental import pallas as pl
from jax.experimental.pallas import tpu as pltpu
from jax.experimental.pallas import tpu_sc as plsc

N = 10000
M = 320000
D = 128
NC = 2
NS = 16
NW = NC * NS
MW = M // NW
CW = 80
CHUNKS = MW // CW
NP = 10240
RPT = NP // NS

_mesh = plsc.VectorSubcoreMesh(
    core_axis_name="c", subcore_axis_name="s", num_cores=NC, num_subcores=NS)

F32 = jnp.float32


def _prep_body(v_ref, w1a_ref, w1b_ref, b1_ref, va_ref, vb_ref):
    v = v_ref[...]
    va_ref[...] = jnp.dot(v, w1a_ref[...], preferred_element_type=F32)
    vb_ref[...] = jnp.dot(v, w1b_ref[...], preferred_element_type=F32) + b1_ref[...]


def _prep(v, w1a, w1b, b1):
    bn = 2000
    grid = N // bn
    return pl.pallas_call(
        _prep_body,
        grid=(grid,),
        in_specs=[
            pl.BlockSpec((bn, D), lambda j: (j, 0)),
            pl.BlockSpec((D, D), lambda j: (0, 0)),
            pl.BlockSpec((D, D), lambda j: (0, 0)),
            pl.BlockSpec((1, D), lambda j: (0, 0)),
        ],
        out_specs=[
            pl.BlockSpec((bn, D), lambda j: (j, 0)),
            pl.BlockSpec((bn, D), lambda j: (j, 0)),
        ],
        out_shape=[jax.ShapeDtypeStruct((N, D), F32)] * 2,
    )(v, w1a, w1b, b1)


def _gather_body(va_hbm, vb_hbm, src_hbm, dst_hbm,
                 sr_out,
                 isrc_v, idst_v, a0, a1, a2, b0, b1, b2,
                 sga0, sga1, sga2, sgb0, sgb1, sgb2, swa0, swa1, swa2):
    c = lax.axis_index("c")
    s = lax.axis_index("s")
    wid = s * NC + c
    wbase = wid * MW

    pltpu.sync_copy(src_hbm.at[wid], isrc_v)
    pltpu.sync_copy(dst_hbm.at[wid], idst_v)

    def isl(iv, j):
        return iv.at[j]

    def osl(j):
        return sr_out.at[pl.ds(wbase + j * CW, CW)]

    def vadd(dst_ref, src_ref):
        def rbody(r, carry):
            for k in range(D // 16):
                sl = pl.ds(k * 16, 16)
                dst_ref[r, sl] = dst_ref[r, sl] + src_ref[r, sl]
            return carry
        lax.fori_loop(0, CW, rbody, 0)

    abufs = (a0, a1, a2)
    bbufs = (b0, b1, b2)
    sgas = (sga0, sga1, sga2)
    sgbs = (sgb0, sgb1, sgb2)
    swas = (swa0, swa1, swa2)
    pltpu.async_copy(va_hbm.at[isl(isrc_v, 0)], a0, sga0)
    pltpu.async_copy(vb_hbm.at[isl(idst_v, 0)], b0, sgb0)
    pltpu.async_copy(va_hbm.at[isl(isrc_v, 1)], a1, sga1)
    pltpu.async_copy(vb_hbm.at[isl(idst_v, 1)], b1, sgb1)

    def step(j, p):
        cur_a, cur_b = abufs[p], bbufs[p]
        n2 = (p + 2) % 3
        nxt_a, nxt_b = abufs[n2], bbufs[n2]
        pltpu.make_async_copy(va_hbm.at[isl(isrc_v, j)], cur_a, sgas[p]).wait()
        pltpu.make_async_copy(vb_hbm.at[isl(idst_v, j)], cur_b, sgbs[p]).wait()

        @pl.when(j + 2 < CHUNKS)
        def _():
            @pl.when(j >= 1)
            def _():
                pltpu.make_async_copy(nxt_a, osl(j - 1), swas[n2]).wait()
            pltpu.async_copy(va_hbm.at[isl(isrc_v, j + 2)], nxt_a, sgas[n2])
            pltpu.async_copy(vb_hbm.at[isl(idst_v, j + 2)], nxt_b, sgbs[n2])

        vadd(cur_a, cur_b)
        pltpu.async_copy(cur_a, osl(j), swas[p])

    def body(j, carry):
        for p in range(3):
            @pl.when(j % 3 == p)
            def _(p=p):
                step(j, p)
        return carry

    lax.fori_loop(0, CHUNKS, body, 0)
    for t in (CHUNKS - 3, CHUNKS - 2, CHUNKS - 1):
        pltpu.make_async_copy(abufs[t % 3], osl(t), swas[t % 3]).wait()


_gather = functools.partial(
    pl.kernel,
    out_type=jax.ShapeDtypeStruct((M, D), F32),
    mesh=_mesh,
    scratch_types=[
        pltpu.VMEM((CHUNKS, CW), jnp.int32),
        pltpu.VMEM((CHUNKS, CW), jnp.int32),
        pltpu.VMEM((CW, D), F32),
        pltpu.VMEM((CW, D), F32),
        pltpu.VMEM((CW, D), F32),
        pltpu.VMEM((CW, D), F32),
        pltpu.VMEM((CW, D), F32),
        pltpu.VMEM((CW, D), F32),
    ] + [pltpu.SemaphoreType.DMA] * 9,
)(_gather_body)


def _counts_body(dst_hbm, zeros_h, ones_h, pcnt_out,
                 idx_v, ones_v, cnt_sh):
    c = lax.axis_index("c")
    s = lax.axis_index("s")
    wid = s * NC + c

    pltpu.sync_copy(zeros_h.at[pl.ds(s * RPT, RPT)],
                    cnt_sh.at[pl.ds(s * RPT, RPT)])
    pltpu.sync_copy(ones_h, ones_v)
    pltpu.sync_copy(dst_hbm.at[wid], idx_v)
    plsc.subcore_barrier()

    def body(j, carry):
        pltpu.sync_copy(ones_v, cnt_sh.at[idx_v.at[j]], add=True)
        return carry

    lax.fori_loop(0, CHUNKS, body, 0)
    plsc.subcore_barrier()
    pltpu.sync_copy(cnt_sh.at[pl.ds(s * RPT, RPT)],
                    pcnt_out.at[c, pl.ds(s * RPT, RPT)])


_counts = functools.partial(
    pl.kernel,
    out_type=jax.ShapeDtypeStruct((NC, NP, D), F32),
    mesh=_mesh,
    scratch_types=[
        pltpu.VMEM((CHUNKS, CW), jnp.int32),
        pltpu.VMEM((CW, D), F32),
        pltpu.VMEM_SHARED((NP, D), F32),
    ],
)(_counts_body)


def _scatter_body(eemb, dstidx, zeros_h, psum,
                  idx_v, r0, r1, r2, acc_sh, sr0, sr1, sr2):
    c = lax.axis_index("c")
    s = lax.axis_index("s")
    wid = s * NC + c
    wbase = wid * MW

    pltpu.sync_copy(zeros_h.at[pl.ds(s * RPT, RPT)],
                    acc_sh.at[pl.ds(s * RPT, RPT)])
    pltpu.sync_copy(dstidx.at[wid], idx_v)
    plsc.subcore_barrier()

    def esl(j):
        return eemb.at[pl.ds(wbase + j * CW, CW)]

    pltpu.async_copy(esl(0), r0, sr0)
    pltpu.async_copy(esl(1), r1, sr1)

    def step(j, cur, nxt2, scur, snxt2):
        pltpu.make_async_copy(esl(j), cur, scur).wait()

        @pl.when(j + 2 < CHUNKS)
        def _():
            pltpu.async_copy(esl(j + 2), nxt2, snxt2)

        pltpu.sync_copy(cur, acc_sh.at[idx_v.at[j]], add=True)

    def body(j, carry):
        @pl.when(j % 3 == 0)
        def _():
            step(j, r0, r2, sr0, sr2)

        @pl.when(j % 3 == 1)
        def _():
            step(j, r1, r0, sr1, sr0)

        @pl.when(j % 3 == 2)
        def _():
            step(j, r2, r1, sr2, sr1)
        return carry

    lax.fori_loop(0, CHUNKS, body, 0)
    plsc.subcore_barrier()
    pltpu.sync_copy(acc_sh.at[pl.ds(s * RPT, RPT)],
                    psum.at[c, pl.ds(s * RPT, RPT)])


_scatter = functools.partial(
    pl.kernel,
    out_type=jax.ShapeDtypeStruct((NC, NP, D), F32),
    mesh=_mesh,
    scratch_types=[
        pltpu.VMEM((CHUNKS, CW), jnp.int32),
        pltpu.VMEM((CW, D), F32),
        pltpu.VMEM((CW, D), F32),
        pltpu.VMEM((CW, D), F32),
        pltpu.VMEM_SHARED((NP, D), F32),
        pltpu.SemaphoreType.DMA,
        pltpu.SemaphoreType.DMA,
        pltpu.SemaphoreType.DMA,
    ],
)(_scatter_body)


def _edge_body(sr_ref, e_ref, w1c_ref, w2_ref, b2_ref, w3_ref, b3_ref,
               out_ref):
    x = sr_ref[...] + jnp.dot(
        e_ref[...], w1c_ref[...], preferred_element_type=F32)
    h1 = jnp.maximum(x, 0.0)
    h2 = jnp.maximum(
        jnp.dot(h1, w2_ref[...], preferred_element_type=F32) + b2_ref[...], 0.0)
    out_ref[...] = jnp.dot(h2, w3_ref[...], preferred_element_type=F32) + b3_ref[...]


def _edge(sr, e, w1c, w2, b2, w3, b3):
    bm = 16000
    grid = M // bm
    wspec = pl.BlockSpec((D, D), lambda j: (0, 0))
    bspec = pl.BlockSpec((1, D), lambda j: (0, 0))
    xspec = pl.BlockSpec((bm, D), lambda j: (j, 0))
    return pl.pallas_call(
        _edge_body,
        grid=(grid,),
        in_specs=[xspec, xspec, wspec, wspec, bspec, wspec, bspec],
        out_specs=xspec,
        out_shape=jax.ShapeDtypeStruct((M, D), F32),
        compiler_params=pltpu.CompilerParams(
            dimension_semantics=("arbitrary",)),
    )(sr, e, w1c, w2, b2, w3, b3)


def _node_body(v_ref, ps_ref, pc_ref, w1a_ref, w1b_ref, b1_ref, w2_ref,
               b2_ref, w3_ref, b3_ref, out_ref):
    ssum = ps_ref[0] + ps_ref[1]
    cnt = pc_ref[0, :, 0:1] + pc_ref[1, :, 0:1]
    mean = ssum / jnp.maximum(cnt, 1.0)
    x = (jnp.dot(v_ref[...], w1a_ref[...], preferred_element_type=F32)
         + jnp.dot(mean, w1b_ref[...], preferred_element_type=F32)
         + b1_ref[...])
    h1 = jnp.maximum(x, 0.0)
    h2 = jnp.maximum(
        jnp.dot(h1, w2_ref[...], preferred_element_type=F32) + b2_ref[...], 0.0)
    out_ref[...] = jnp.dot(h2, w3_ref[...], preferred_element_type=F32) + b3_ref[...]


def _node(v, psum, pcnt, w1a, w1b, b1, w2, b2, w3, b3):
    bn = 2000
    grid = N // bn
    wspec = pl.BlockSpec((D, D), lambda j: (0, 0))
    bspec = pl.BlockSpec((1, D), lambda j: (0, 0))
    return pl.pallas_call(
        _node_body,
        grid=(grid,),
        in_specs=[
            pl.BlockSpec((bn, D), lambda j: (j, 0)),
            pl.BlockSpec((NC, bn, D), lambda j: (0, j, 0)),
            pl.BlockSpec((NC, bn, D), lambda j: (0, j, 0)),
            wspec, wspec, bspec, wspec, bspec, wspec, bspec,
        ],
        out_specs=pl.BlockSpec((bn, D), lambda j: (j, 0)),
        out_shape=jax.ShapeDtypeStruct((N, D), F32),
    )(v, psum, pcnt, w1a, w1b, b1, w2, b2, w3, b3)


def kernel(V, E, edges, fe_W1, fe_b1, fe_W2, fe_b2, fe_W3, fe_b3,
           fn_W1, fn_b1, fn_W2, fn_b2, fn_W3, fn_b3):
    v2 = V[0]
    e2 = E[0]
    edges = edges.astype(jnp.int32)
    src = edges[0, :, 0].reshape(NW, CHUNKS, CW)
    dst = edges[0, :, 1].reshape(NW, CHUNKS, CW)

    zeros_h = jnp.zeros((NP, D), F32)
    ones_h = jnp.ones((CW, D), F32)

    va, vb = _prep(v2, fe_W1[:D], fe_W1[D:2 * D], fe_b1.reshape(1, D))
    sr_rows = _gather(va, vb, src, dst)
    pcnt = _counts(dst, zeros_h, ones_h)
    eemb = _edge(sr_rows, e2, fe_W1[2 * D:], fe_W2,
                 fe_b2.reshape(1, D), fe_W3, fe_b3.reshape(1, D))

    psum = _scatter(eemb, dst, zeros_h)

    nemb = _node(v2, psum[:, :N], pcnt[:, :N],
                 fn_W1[:D], fn_W1[D:], fn_b1.reshape(1, D),
                 fn_W2, fn_b2.reshape(1, D), fn_W3, fn_b3.reshape(1, D))
    return (nemb[None], eemb[None])
